# Initial kernel scaffold; baseline (speedup 1.0000x reference)
import jax
import jax.numpy as jnp
from jax.experimental import pallas as pl


def kernel(x, up_index, boundary_index, W1u, b1u, g1u, be1u, W2u, b2u, g2u, be2u, W1b, b1b, g1b, be1b, W2b, b2b, g2b, be2b, Wc, bc, gc, bec):
    raise NotImplementedError("write your pallas kernel here")


# speedup vs baseline: 7.4060x; 7.4060x over previous
import functools

import jax
import jax.numpy as jnp
from jax import lax
from jax.experimental import pallas as pl
from jax.experimental.pallas import tpu as pltpu
from jax.experimental.pallas import tpu_sc as plsc

_NC = 2
_NS = 16
_C = 256
_PAD_ROWS = 16


def _sc_scatter(x, srcs, dsts, n, d, e_pad):
    chunks_per_tile = e_pad // (_C * _NS)
    rows_per_tile = (n // _NS) // 8 * 8
    tail_rows = n - _NS * rows_per_tile

    mesh = plsc.VectorSubcoreMesh(core_axis_name="c", subcore_axis_name="s")

    @functools.partial(
        pl.kernel,
        mesh=mesh,
        out_type=jax.ShapeDtypeStruct((_NC, n, d), jnp.float32),
        scratch_types=[
            pltpu.VMEM_SHARED((n + _PAD_ROWS, d), jnp.float32),
            pltpu.VMEM((_C,), jnp.int32),
            pltpu.VMEM((_C,), jnp.int32),
            pltpu.VMEM((_C, d), jnp.float32),
            pltpu.SemaphoreType.DMA,
        ],
    )
    def k(x_hbm, srcs_hbm, dsts_hbm, out_hbm, acc_sh, src_v, dst_v, rows_v, sem):
        c = lax.axis_index("c")
        s = lax.axis_index("s")
        r0 = s * rows_per_tile
        pltpu.sync_copy(x_hbm.at[pl.ds(r0, rows_per_tile)],
                        acc_sh.at[pl.ds(r0, rows_per_tile)])
        if tail_rows:
            @pl.when(s == 0)
            def _():
                t0 = _NS * rows_per_tile
                pltpu.sync_copy(x_hbm.at[pl.ds(t0, tail_rows)],
                                acc_sh.at[pl.ds(t0, tail_rows)])
        plsc.subcore_barrier()

        edge0 = c * e_pad + s * chunks_per_tile * _C

        def body(i, carry):
            e0 = edge0 + i * _C
            pltpu.sync_copy(srcs_hbm.at[pl.ds(e0, _C)], src_v)
            pltpu.sync_copy(dsts_hbm.at[pl.ds(e0, _C)], dst_v)
            pltpu.async_copy(x_hbm.at[src_v], rows_v, sem).wait()
            pltpu.sync_copy(rows_v, acc_sh.at[dst_v], add=True)
            return carry

        lax.fori_loop(0, chunks_per_tile, body, 0)
        plsc.subcore_barrier()
        pltpu.sync_copy(acc_sh.at[pl.ds(r0, rows_per_tile)],
                        out_hbm.at[c, pl.ds(r0, rows_per_tile)])
        if tail_rows:
            @pl.when(s == 0)
            def _():
                t0 = _NS * rows_per_tile
                pltpu.sync_copy(acc_sh.at[pl.ds(t0, tail_rows)],
                                out_hbm.at[c, pl.ds(t0, tail_rows)])

    return k(x, srcs, dsts)


def _bn(h, g, b):
    m = jnp.mean(h, axis=0)
    v = jnp.var(h, axis=0)
    return (h - m) * lax.rsqrt(v + 1e-5) * g + b


def _mlp_body(acc_ref, W1u, b1u, g1u, be1u, W2u, b2u, g2u, be2u,
              W1b, b1b, g1b, be1b, W2b, b2b, g2b, be2b,
              Wc, bc, gc, bec, out_ref):
    h = acc_ref[0]
    hu = jax.nn.relu(_bn(jnp.dot(h, W1u[...], preferred_element_type=jnp.float32)
                         + b1u[...], g1u[...], be1u[...]))
    hu = jax.nn.relu(_bn(jnp.dot(hu, W2u[...], preferred_element_type=jnp.float32)
                         + b2u[...], g2u[...], be2u[...]))
    h = acc_ref[1]
    hb = jax.nn.relu(_bn(jnp.dot(h, W1b[...], preferred_element_type=jnp.float32)
                         + b1b[...], g1b[...], be1b[...]))
    hb = jax.nn.relu(_bn(jnp.dot(hb, W2b[...], preferred_element_type=jnp.float32)
                         + b2b[...], g2b[...], be2b[...]))
    hh = Wc.shape[0] // 2
    hc = (jnp.dot(hu, Wc[0:hh, :], preferred_element_type=jnp.float32)
          + jnp.dot(hb, Wc[hh:, :], preferred_element_type=jnp.float32))
    out_ref[...] = jax.nn.relu(_bn(hc + bc[...], gc[...], bec[...]))


def kernel(x, up_index, boundary_index, W1u, b1u, g1u, be1u, W2u, b2u, g2u, be2u,
           W1b, b1b, g1b, be1b, W2b, b2b, g2b, be2b, Wc, bc, gc, bec):
    n, d = x.shape
    e = up_index.shape[1]
    per_tile = _C * _NS
    e_pad = ((e + per_tile - 1) // per_tile) * per_tile
    pad = e_pad - e

    pad_src = (jnp.arange(pad, dtype=jnp.int32) * 131) % n
    pad_dst = n + (jnp.arange(pad, dtype=jnp.int32) % _PAD_ROWS)
    i32 = jnp.int32
    srcs = jnp.stack([
        jnp.concatenate([up_index[0].astype(i32), pad_src]),
        jnp.concatenate([boundary_index[0].astype(i32), pad_src]),
    ]).reshape(-1)
    dsts = jnp.stack([
        jnp.concatenate([up_index[1].astype(i32), pad_dst]),
        jnp.concatenate([boundary_index[1].astype(i32), pad_dst]),
    ]).reshape(-1)

    acc = _sc_scatter(x, srcs, dsts, n, d, e_pad)

    h = Wc.shape[1]
    return pl.pallas_call(
        _mlp_body,
        out_shape=jax.ShapeDtypeStruct((n, h), jnp.float32),
    )(acc, W1u, b1u, g1u, be1u, W2u, b2u, g2u, be2u,
      W1b, b1b, g1b, be1b, W2b, b2b, g2b, be2b, Wc, bc, gc, bec)


# speedup vs baseline: 10.5417x; 1.4234x over previous
import functools

import jax
import jax.numpy as jnp
from jax import lax
from jax.experimental import pallas as pl
from jax.experimental.pallas import tpu as pltpu
from jax.experimental.pallas import tpu_sc as plsc

_NC = 2
_NS = 16
_C = 128
_G = 16
_PAD_ROWS = 16


def _sc_scatter(x, srcs, dsts, n, d, e_pad):
    chunks_per_tile = e_pad // (_C * _NS)
    groups_per_tile = chunks_per_tile // _G
    rows_per_tile = (n // _NS) // 8 * 8
    tail_rows = n - _NS * rows_per_tile

    mesh = plsc.VectorSubcoreMesh(core_axis_name="c", subcore_axis_name="s")

    @functools.partial(
        pl.kernel,
        mesh=mesh,
        out_type=jax.ShapeDtypeStruct((_NC, n, d), jnp.float32),
        scratch_types=[
            pltpu.VMEM_SHARED((n + _PAD_ROWS, d), jnp.float32),
            pltpu.VMEM((_G, _C), jnp.int32),
            pltpu.VMEM((_G, _C), jnp.int32),
            pltpu.VMEM((2, _C, d), jnp.float32),
            pltpu.SemaphoreType.DMA,
            pltpu.SemaphoreType.DMA,
            pltpu.SemaphoreType.DMA,
            pltpu.SemaphoreType.DMA,
        ],
    )
    def k(x_hbm, srcs_hbm, dsts_hbm, out_hbm, acc_sh, src_v, dst_v, rows_v,
          sem_g0, sem_g1, sem_s0, sem_s1):
        sem_g = [sem_g0, sem_g1]
        sem_s = [sem_s0, sem_s1]
        c = lax.axis_index("c")
        s = lax.axis_index("s")
        r0 = s * rows_per_tile
        pltpu.sync_copy(x_hbm.at[pl.ds(r0, rows_per_tile)],
                        acc_sh.at[pl.ds(r0, rows_per_tile)])
        if tail_rows:
            @pl.when(s == 0)
            def _():
                t0 = _NS * rows_per_tile
                pltpu.sync_copy(x_hbm.at[pl.ds(t0, tail_rows)],
                                acc_sh.at[pl.ds(t0, tail_rows)])
        plsc.subcore_barrier()

        row0 = c * (e_pad // _C) + s * chunks_per_tile

        def group(g, carry):
            gr0 = row0 + g * _G
            pltpu.sync_copy(srcs_hbm.at[pl.ds(gr0, _G)], src_v)
            pltpu.sync_copy(dsts_hbm.at[pl.ds(gr0, _G)], dst_v)
            desc_g = [None, None]
            desc_s = [None, None]
            for j in range(_G):
                b = j % 2
                if j >= 2:
                    desc_s[b].wait()
                desc_g[b] = pltpu.async_copy(x_hbm.at[src_v.at[j]],
                                             rows_v.at[b], sem_g[b])
                if j >= 1:
                    desc_g[1 - b].wait()
                    desc_s[1 - b] = pltpu.async_copy(
                        rows_v.at[1 - b], acc_sh.at[dst_v.at[j - 1]],
                        sem_s[1 - b], add=True)
            last = (_G - 1) % 2
            desc_g[last].wait()
            desc_s[last] = pltpu.async_copy(
                rows_v.at[last], acc_sh.at[dst_v.at[_G - 1]],
                sem_s[last], add=True)
            desc_s[1 - last].wait()
            desc_s[last].wait()
            return carry

        lax.fori_loop(0, groups_per_tile, group, 0)
        plsc.subcore_barrier()
        pltpu.sync_copy(acc_sh.at[pl.ds(r0, rows_per_tile)],
                        out_hbm.at[c, pl.ds(r0, rows_per_tile)])
        if tail_rows:
            @pl.when(s == 0)
            def _():
                t0 = _NS * rows_per_tile
                pltpu.sync_copy(acc_sh.at[pl.ds(t0, tail_rows)],
                                out_hbm.at[c, pl.ds(t0, tail_rows)])

    return k(x, srcs, dsts)


def _bn(h, g, b):
    m = jnp.mean(h, axis=0)
    v = jnp.var(h, axis=0)
    return (h - m) * lax.rsqrt(v + 1e-5) * g + b


def _mlp_body(acc_ref, W1u, b1u, g1u, be1u, W2u, b2u, g2u, be2u,
              W1b, b1b, g1b, be1b, W2b, b2b, g2b, be2b,
              Wc, bc, gc, bec, out_ref):
    h = acc_ref[0]
    hu = jax.nn.relu(_bn(jnp.dot(h, W1u[...], preferred_element_type=jnp.float32)
                         + b1u[...], g1u[...], be1u[...]))
    hu = jax.nn.relu(_bn(jnp.dot(hu, W2u[...], preferred_element_type=jnp.float32)
                         + b2u[...], g2u[...], be2u[...]))
    h = acc_ref[1]
    hb = jax.nn.relu(_bn(jnp.dot(h, W1b[...], preferred_element_type=jnp.float32)
                         + b1b[...], g1b[...], be1b[...]))
    hb = jax.nn.relu(_bn(jnp.dot(hb, W2b[...], preferred_element_type=jnp.float32)
                         + b2b[...], g2b[...], be2b[...]))
    hh = Wc.shape[0] // 2
    hc = (jnp.dot(hu, Wc[0:hh, :], preferred_element_type=jnp.float32)
          + jnp.dot(hb, Wc[hh:, :], preferred_element_type=jnp.float32))
    out_ref[...] = jax.nn.relu(_bn(hc + bc[...], gc[...], bec[...]))


def kernel(x, up_index, boundary_index, W1u, b1u, g1u, be1u, W2u, b2u, g2u, be2u,
           W1b, b1b, g1b, be1b, W2b, b2b, g2b, be2b, Wc, bc, gc, bec):
    n, d = x.shape
    e = up_index.shape[1]
    per_round = _C * _NS * _G
    e_pad = ((e + per_round - 1) // per_round) * per_round
    pad = e_pad - e

    pad_src = (jnp.arange(pad, dtype=jnp.int32) * 131) % n
    pad_dst = n + (jnp.arange(pad, dtype=jnp.int32) % _PAD_ROWS)
    i32 = jnp.int32
    srcs = jnp.stack([
        jnp.concatenate([up_index[0].astype(i32), pad_src]),
        jnp.concatenate([boundary_index[0].astype(i32), pad_src]),
    ]).reshape(-1, _C)
    dsts = jnp.stack([
        jnp.concatenate([up_index[1].astype(i32), pad_dst]),
        jnp.concatenate([boundary_index[1].astype(i32), pad_dst]),
    ]).reshape(-1, _C)

    acc = _sc_scatter(x, srcs, dsts, n, d, e_pad)

    h = Wc.shape[1]
    return pl.pallas_call(
        _mlp_body,
        out_shape=jax.ShapeDtypeStruct((n, h), jnp.float32),
    )(acc, W1u, b1u, g1u, be1u, W2u, b2u, g2u, be2u,
      W1b, b1b, g1b, be1b, W2b, b2b, g2b, be2b, Wc, bc, gc, bec)


# speedup vs baseline: 10.8354x; 1.0279x over previous
import functools

import jax
import jax.numpy as jnp
from jax import lax
from jax.experimental import pallas as pl
from jax.experimental.pallas import tpu as pltpu
from jax.experimental.pallas import tpu_sc as plsc

_NC = 2
_NS = 16
_C = 128
_G = 32
_PAD_ROWS = 16


def _sc_scatter(x, srcs, dsts, n, d, e_pad):
    chunks_per_tile = e_pad // (_C * _NS)
    groups_per_tile = chunks_per_tile // _G
    rows_per_tile = (n // _NS) // 8 * 8
    tail_rows = n - _NS * rows_per_tile

    mesh = plsc.VectorSubcoreMesh(core_axis_name="c", subcore_axis_name="s")

    @functools.partial(
        pl.kernel,
        mesh=mesh,
        out_type=jax.ShapeDtypeStruct((_NC, n, d), jnp.float32),
        scratch_types=[
            pltpu.VMEM_SHARED((n + _PAD_ROWS, d), jnp.float32),
            pltpu.VMEM((_G, _C), jnp.int32),
            pltpu.VMEM((_G, _C), jnp.int32),
            pltpu.VMEM((2, _C, d), jnp.float32),
            pltpu.SemaphoreType.DMA,
            pltpu.SemaphoreType.DMA,
            pltpu.SemaphoreType.DMA,
            pltpu.SemaphoreType.DMA,
        ],
    )
    def k(x_hbm, srcs_hbm, dsts_hbm, out_hbm, acc_sh, src_v, dst_v, rows_v,
          sem_g0, sem_g1, sem_s0, sem_s1):
        sem_g = [sem_g0, sem_g1]
        sem_s = [sem_s0, sem_s1]
        c = lax.axis_index("c")
        s = lax.axis_index("s")
        r0 = s * rows_per_tile
        pltpu.sync_copy(x_hbm.at[pl.ds(r0, rows_per_tile)],
                        acc_sh.at[pl.ds(r0, rows_per_tile)])
        if tail_rows:
            @pl.when(s == 0)
            def _():
                t0 = _NS * rows_per_tile
                pltpu.sync_copy(x_hbm.at[pl.ds(t0, tail_rows)],
                                acc_sh.at[pl.ds(t0, tail_rows)])
        plsc.subcore_barrier()

        row0 = c * (e_pad // _C) + s * chunks_per_tile

        def group(g, carry):
            gr0 = row0 + g * _G
            pltpu.sync_copy(srcs_hbm.at[pl.ds(gr0, _G)], src_v)
            pltpu.sync_copy(dsts_hbm.at[pl.ds(gr0, _G)], dst_v)
            desc_g = [None, None]
            desc_s = [None, None]
            for j in range(_G):
                b = j % 2
                if j >= 2:
                    desc_s[b].wait()
                desc_g[b] = pltpu.async_copy(x_hbm.at[src_v.at[j]],
                                             rows_v.at[b], sem_g[b])
                if j >= 1:
                    desc_g[1 - b].wait()
                    desc_s[1 - b] = pltpu.async_copy(
                        rows_v.at[1 - b], acc_sh.at[dst_v.at[j - 1]],
                        sem_s[1 - b], add=True)
            last = (_G - 1) % 2
            desc_g[last].wait()
            desc_s[last] = pltpu.async_copy(
                rows_v.at[last], acc_sh.at[dst_v.at[_G - 1]],
                sem_s[last], add=True)
            desc_s[1 - last].wait()
            desc_s[last].wait()
            return carry

        lax.fori_loop(0, groups_per_tile, group, 0)
        plsc.subcore_barrier()
        pltpu.sync_copy(acc_sh.at[pl.ds(r0, rows_per_tile)],
                        out_hbm.at[c, pl.ds(r0, rows_per_tile)])
        if tail_rows:
            @pl.when(s == 0)
            def _():
                t0 = _NS * rows_per_tile
                pltpu.sync_copy(acc_sh.at[pl.ds(t0, tail_rows)],
                                out_hbm.at[c, pl.ds(t0, tail_rows)])

    return k(x, srcs, dsts)


def _bn(h, g, b):
    m = jnp.mean(h, axis=0)
    v = jnp.var(h, axis=0)
    return (h - m) * lax.rsqrt(v + 1e-5) * g + b


def _mlp_body(acc_ref, W1u, b1u, g1u, be1u, W2u, b2u, g2u, be2u,
              W1b, b1b, g1b, be1b, W2b, b2b, g2b, be2b,
              Wc, bc, gc, bec, out_ref):
    h = acc_ref[0]
    hu = jax.nn.relu(_bn(jnp.dot(h, W1u[...], preferred_element_type=jnp.float32)
                         + b1u[...], g1u[...], be1u[...]))
    hu = jax.nn.relu(_bn(jnp.dot(hu, W2u[...], preferred_element_type=jnp.float32)
                         + b2u[...], g2u[...], be2u[...]))
    h = acc_ref[1]
    hb = jax.nn.relu(_bn(jnp.dot(h, W1b[...], preferred_element_type=jnp.float32)
                         + b1b[...], g1b[...], be1b[...]))
    hb = jax.nn.relu(_bn(jnp.dot(hb, W2b[...], preferred_element_type=jnp.float32)
                         + b2b[...], g2b[...], be2b[...]))
    hh = Wc.shape[0] // 2
    hc = (jnp.dot(hu, Wc[0:hh, :], preferred_element_type=jnp.float32)
          + jnp.dot(hb, Wc[hh:, :], preferred_element_type=jnp.float32))
    out_ref[...] = jax.nn.relu(_bn(hc + bc[...], gc[...], bec[...]))


def kernel(x, up_index, boundary_index, W1u, b1u, g1u, be1u, W2u, b2u, g2u, be2u,
           W1b, b1b, g1b, be1b, W2b, b2b, g2b, be2b, Wc, bc, gc, bec):
    n, d = x.shape
    e = up_index.shape[1]
    per_round = _C * _NS * _G
    e_pad = ((e + per_round - 1) // per_round) * per_round
    pad = e_pad - e

    pad_src = (jnp.arange(pad, dtype=jnp.int32) * 131) % n
    pad_dst = n + (jnp.arange(pad, dtype=jnp.int32) % _PAD_ROWS)
    i32 = jnp.int32
    srcs = jnp.stack([
        jnp.concatenate([up_index[0].astype(i32), pad_src]),
        jnp.concatenate([boundary_index[0].astype(i32), pad_src]),
    ]).reshape(-1, _C)
    dsts = jnp.stack([
        jnp.concatenate([up_index[1].astype(i32), pad_dst]),
        jnp.concatenate([boundary_index[1].astype(i32), pad_dst]),
    ]).reshape(-1, _C)

    acc = _sc_scatter(x, srcs, dsts, n, d, e_pad)

    h = Wc.shape[1]
    return pl.pallas_call(
        _mlp_body,
        out_shape=jax.ShapeDtypeStruct((n, h), jnp.float32),
    )(acc, W1u, b1u, g1u, be1u, W2u, b2u, g2u, be2u,
      W1b, b1b, g1b, be1b, W2b, b2b, g2b, be2b, Wc, bc, gc, bec)


# speedup vs baseline: 12.2860x; 1.1339x over previous
import functools

import jax
import jax.numpy as jnp
from jax import lax
from jax.experimental import pallas as pl
from jax.experimental.pallas import tpu as pltpu
from jax.experimental.pallas import tpu_sc as plsc

_NC = 2
_NS = 16
_C = 128
_G = 32
_PAD_ROWS = 16


def _sc_scatter(x, idxr, n, d, ec):
    q8, r8 = divmod(ec // 8, _NS)
    cpt_big = (q8 + 1) * 8
    cpt_small = q8 * 8
    rem_c = ec % 8
    gf_big, lo_big = divmod(cpt_big, _G)
    gf_small, lo_small = divmod(cpt_small, _G)

    rows_per_tile = (n // _NS) // 8 * 8
    tail_rows = n - _NS * rows_per_tile

    mesh = plsc.VectorSubcoreMesh(core_axis_name="c", subcore_axis_name="s")

    @functools.partial(
        pl.kernel,
        mesh=mesh,
        out_type=jax.ShapeDtypeStruct((_NC, n, d), jnp.float32),
        scratch_types=[
            pltpu.VMEM_SHARED((n + _PAD_ROWS, d), jnp.float32),
            pltpu.VMEM((_G, _C), jnp.int32),
            pltpu.VMEM((_G, _C), jnp.int32),
            pltpu.VMEM((2, _C, d), jnp.float32),
            pltpu.SemaphoreType.DMA,
            pltpu.SemaphoreType.DMA,
            pltpu.SemaphoreType.DMA,
            pltpu.SemaphoreType.DMA,
        ],
    )
    def k(x_hbm, idx_hbm, out_hbm, acc_sh, src_v, dst_v, rows_v,
          sem_g0, sem_g1, sem_s0, sem_s1):
        sem_g = [sem_g0, sem_g1]
        sem_s = [sem_s0, sem_s1]
        c = lax.axis_index("c")
        s = lax.axis_index("s")
        r0 = s * rows_per_tile
        pltpu.sync_copy(x_hbm.at[pl.ds(r0, rows_per_tile)],
                        acc_sh.at[pl.ds(r0, rows_per_tile)])
        if tail_rows:
            @pl.when(s == 0)
            def _():
                t0 = _NS * rows_per_tile
                pltpu.sync_copy(x_hbm.at[pl.ds(t0, tail_rows)],
                                acc_sh.at[pl.ds(t0, tail_rows)])
        plsc.subcore_barrier()

        chunk0 = pl.multiple_of(
            jnp.where(s < r8, s * cpt_big,
                      r8 * cpt_big + (s - r8) * cpt_small), 8)

        def load_idx(row, nrows):
            row = pl.multiple_of(row, 8)
            pltpu.sync_copy(idx_hbm.at[c, 0, pl.ds(row, nrows)],
                            src_v.at[pl.ds(0, nrows)])
            pltpu.sync_copy(idx_hbm.at[c, 1, pl.ds(row, nrows)],
                            dst_v.at[pl.ds(0, nrows)])

        def gather(j, b):
            return pltpu.async_copy(x_hbm.at[src_v.at[j]], rows_v.at[b],
                                    sem_g[b])

        def scatter(j, b):
            return pltpu.async_copy(rows_v.at[b], acc_sh.at[dst_v.at[j]],
                                    sem_s[b], add=True)

        def pipe(row0, nchunks):
            load_idx(row0, nchunks)
            desc_g = [None, None]
            desc_s = [None, None]
            for j in range(nchunks):
                b = j % 2
                if j >= 2:
                    desc_s[b].wait()
                desc_g[b] = gather(j, b)
                if j >= 1:
                    desc_g[1 - b].wait()
                    desc_s[1 - b] = scatter(j - 1, 1 - b)
            last = (nchunks - 1) % 2
            desc_g[last].wait()
            desc_s[last] = scatter(nchunks - 1, last)
            if nchunks >= 2:
                desc_s[1 - last].wait()
            desc_s[last].wait()

        def group(g, carry):
            pipe(chunk0 + g * _G, _G)
            return carry

        gf_dyn = jnp.where(s < r8, gf_big, gf_small)
        lax.fori_loop(0, gf_dyn, group, 0)

        if lo_big:
            @pl.when(s < r8)
            def _():
                pipe(chunk0 + gf_big * _G, lo_big)
        if lo_small:
            @pl.when(s >= r8)
            def _():
                pipe(chunk0 + gf_small * _G, lo_small)
        if rem_c:
            @pl.when(s == _NS - 1)
            def _():
                pipe(ec - rem_c, rem_c)

        plsc.subcore_barrier()
        pltpu.sync_copy(acc_sh.at[pl.ds(r0, rows_per_tile)],
                        out_hbm.at[c, pl.ds(r0, rows_per_tile)])
        if tail_rows:
            @pl.when(s == 0)
            def _():
                t0 = _NS * rows_per_tile
                pltpu.sync_copy(acc_sh.at[pl.ds(t0, tail_rows)],
                                out_hbm.at[c, pl.ds(t0, tail_rows)])

    return k(x, idxr)


def _bn(h, g, b):
    m = jnp.mean(h, axis=0)
    v = jnp.var(h, axis=0)
    return (h - m) * lax.rsqrt(v + 1e-5) * g + b


def _mlp_body(acc_ref, W1u, b1u, g1u, be1u, W2u, b2u, g2u, be2u,
              W1b, b1b, g1b, be1b, W2b, b2b, g2b, be2b,
              Wc, bc, gc, bec, out_ref):
    h = acc_ref[0]
    hu = jax.nn.relu(_bn(jnp.dot(h, W1u[...], preferred_element_type=jnp.float32)
                         + b1u[...], g1u[...], be1u[...]))
    hu = jax.nn.relu(_bn(jnp.dot(hu, W2u[...], preferred_element_type=jnp.float32)
                         + b2u[...], g2u[...], be2u[...]))
    h = acc_ref[1]
    hb = jax.nn.relu(_bn(jnp.dot(h, W1b[...], preferred_element_type=jnp.float32)
                         + b1b[...], g1b[...], be1b[...]))
    hb = jax.nn.relu(_bn(jnp.dot(hb, W2b[...], preferred_element_type=jnp.float32)
                         + b2b[...], g2b[...], be2b[...]))
    hh = Wc.shape[0] // 2
    hc = (jnp.dot(hu, Wc[0:hh, :], preferred_element_type=jnp.float32)
          + jnp.dot(hb, Wc[hh:, :], preferred_element_type=jnp.float32))
    out_ref[...] = jax.nn.relu(_bn(hc + bc[...], gc[...], bec[...]))


def kernel(x, up_index, boundary_index, W1u, b1u, g1u, be1u, W2u, b2u, g2u, be2u,
           W1b, b1b, g1b, be1b, W2b, b2b, g2b, be2b, Wc, bc, gc, bec):
    n, d = x.shape
    e = up_index.shape[1]
    i32 = jnp.int32
    up = up_index.astype(i32)
    bd = boundary_index.astype(i32)
    if e % _C:
        pad = _C - e % _C
        psrc = jnp.zeros((1, pad), i32)
        pdst = jnp.full((1, pad), n, i32)
        up = jnp.concatenate([up, jnp.concatenate([psrc, pdst])], axis=1)
        bd = jnp.concatenate([bd, jnp.concatenate([psrc, pdst])], axis=1)
        e += pad
    ec = e // _C
    idxr = jnp.stack([up, bd]).reshape(_NC, 2, ec, _C)

    acc = _sc_scatter(x, idxr, n, d, ec)

    h = Wc.shape[1]
    return pl.pallas_call(
        _mlp_body,
        out_shape=jax.ShapeDtypeStruct((n, h), jnp.float32),
    )(acc, W1u, b1u, g1u, be1u, W2u, b2u, g2u, be2u,
      W1b, b1b, g1b, be1b, W2b, b2b, g2b, be2b, Wc, bc, gc, bec)


# speedup vs baseline: 12.4294x; 1.0117x over previous
import functools

import jax
import jax.numpy as jnp
from jax import lax
from jax.experimental import pallas as pl
from jax.experimental.pallas import tpu as pltpu
from jax.experimental.pallas import tpu_sc as plsc

_NC = 2
_NS = 16
_C = 128
_G = 32
_PAD_ROWS = 16


def _sc_scatter(x, idxr, n, d, ec):
    q8, r8 = divmod(ec // 8, _NS)
    cpt_big = (q8 + 1) * 8
    cpt_small = q8 * 8
    rem_c = ec % 8
    gf_big, lo_big = divmod(cpt_big, _G)
    gf_small, lo_small = divmod(cpt_small, _G)

    rows_per_tile = (n // _NS) // 16 * 16
    tail_rows = n - _NS * rows_per_tile

    mesh = plsc.VectorSubcoreMesh(core_axis_name="c", subcore_axis_name="s")

    @functools.partial(
        pl.kernel,
        mesh=mesh,
        out_type=jax.ShapeDtypeStruct((_NC, n, d), jnp.float32),
        scratch_types=[
            pltpu.VMEM_SHARED((n + _PAD_ROWS, d), jnp.float32),
            pltpu.VMEM((_G, _C), jnp.int32),
            pltpu.VMEM((_G, _C), jnp.int32),
            pltpu.VMEM((2, _C, d), jnp.float32),
            pltpu.SemaphoreType.DMA,
            pltpu.SemaphoreType.DMA,
            pltpu.SemaphoreType.DMA,
            pltpu.SemaphoreType.DMA,
        ],
    )
    def k(x_hbm, idx_hbm, out_hbm, acc_sh, src_v, dst_v, rows_v,
          sem_g0, sem_g1, sem_s0, sem_s1):
        sem_g = [sem_g0, sem_g1]
        sem_s = [sem_s0, sem_s1]
        c = lax.axis_index("c")
        s = lax.axis_index("s")
        r0 = s * rows_per_tile
        pltpu.sync_copy(x_hbm.at[pl.ds(r0, rows_per_tile)],
                        acc_sh.at[pl.ds(r0, rows_per_tile)])
        if tail_rows:
            @pl.when(s == 0)
            def _():
                t0 = _NS * rows_per_tile
                pltpu.sync_copy(x_hbm.at[pl.ds(t0, tail_rows)],
                                acc_sh.at[pl.ds(t0, tail_rows)])
        plsc.subcore_barrier()

        chunk0 = pl.multiple_of(
            jnp.where(s < r8, s * cpt_big,
                      r8 * cpt_big + (s - r8) * cpt_small), 8)

        def load_idx(row, nrows):
            row = pl.multiple_of(row, 8)
            pltpu.sync_copy(idx_hbm.at[c, 0, pl.ds(row, nrows)],
                            src_v.at[pl.ds(0, nrows)])
            pltpu.sync_copy(idx_hbm.at[c, 1, pl.ds(row, nrows)],
                            dst_v.at[pl.ds(0, nrows)])

        def gather(j, b):
            return pltpu.async_copy(x_hbm.at[src_v.at[j]], rows_v.at[b],
                                    sem_g[b])

        def scatter(j, b):
            return pltpu.async_copy(rows_v.at[b], acc_sh.at[dst_v.at[j]],
                                    sem_s[b], add=True)

        def pipe(row0, nchunks):
            load_idx(row0, nchunks)
            desc_g = [None, None]
            desc_s = [None, None]
            for j in range(nchunks):
                b = j % 2
                if j >= 2:
                    desc_s[b].wait()
                desc_g[b] = gather(j, b)
                if j >= 1:
                    desc_g[1 - b].wait()
                    desc_s[1 - b] = scatter(j - 1, 1 - b)
            last = (nchunks - 1) % 2
            desc_g[last].wait()
            desc_s[last] = scatter(nchunks - 1, last)
            if nchunks >= 2:
                desc_s[1 - last].wait()
            desc_s[last].wait()

        def group(g, carry):
            pipe(chunk0 + g * _G, _G)
            return carry

        gf_dyn = jnp.where(s < r8, gf_big, gf_small)
        lax.fori_loop(0, gf_dyn, group, 0)

        if lo_big:
            @pl.when(s < r8)
            def _():
                pipe(chunk0 + gf_big * _G, lo_big)
        if lo_small:
            @pl.when(s >= r8)
            def _():
                pipe(chunk0 + gf_small * _G, lo_small)
        if rem_c:
            @pl.when(s == _NS - 1)
            def _():
                pipe(ec - rem_c, rem_c)

        plsc.subcore_barrier()
        pltpu.sync_copy(acc_sh.at[pl.ds(r0, rows_per_tile)],
                        out_hbm.at[c, pl.ds(r0, rows_per_tile)])
        if tail_rows:
            @pl.when(s == 0)
            def _():
                t0 = _NS * rows_per_tile
                pltpu.sync_copy(acc_sh.at[pl.ds(t0, tail_rows)],
                                out_hbm.at[c, pl.ds(t0, tail_rows)])

    return k(x, idxr)


def _bn(h, g, b):
    m = jnp.mean(h, axis=0)
    v = jnp.mean(h * h, axis=0) - m * m
    return (h - m) * (lax.rsqrt(v + 1e-5) * g) + b


def _dot(a, w):
    return jnp.dot(a, w, preferred_element_type=jnp.float32)


def _mlp_body(acc_ref, W1u, b1u, g1u, be1u, W2u, b2u, g2u, be2u,
              W1b, b1b, g1b, be1b, W2b, b2b, g2b, be2b,
              Wc, bc, gc, bec, out_ref):
    h = acc_ref[0]
    hu = jax.nn.relu(_bn(_dot(h, W1u[...]) + b1u[...], g1u[...], be1u[...]))
    hu = jax.nn.relu(_bn(_dot(hu, W2u[...]) + b2u[...], g2u[...], be2u[...]))
    h = acc_ref[1]
    hb = jax.nn.relu(_bn(_dot(h, W1b[...]) + b1b[...], g1b[...], be1b[...]))
    hb = jax.nn.relu(_bn(_dot(hb, W2b[...]) + b2b[...], g2b[...], be2b[...]))
    hh = Wc.shape[0] // 2
    hc = _dot(hu, Wc[0:hh, :]) + _dot(hb, Wc[hh:, :])
    out_ref[...] = jax.nn.relu(_bn(hc + bc[...], gc[...], bec[...]))


def kernel(x, up_index, boundary_index, W1u, b1u, g1u, be1u, W2u, b2u, g2u, be2u,
           W1b, b1b, g1b, be1b, W2b, b2b, g2b, be2b, Wc, bc, gc, bec):
    n, d = x.shape
    e = up_index.shape[1]
    i32 = jnp.int32
    up = up_index.astype(i32)
    bd = boundary_index.astype(i32)
    if e % _C:
        pad = _C - e % _C
        psrc = jnp.zeros((1, pad), i32)
        pdst = jnp.full((1, pad), n, i32)
        up = jnp.concatenate([up, jnp.concatenate([psrc, pdst])], axis=1)
        bd = jnp.concatenate([bd, jnp.concatenate([psrc, pdst])], axis=1)
        e += pad
    ec = e // _C
    idxr = jnp.stack([up, bd]).reshape(_NC, 2, ec, _C)

    acc = _sc_scatter(x, idxr, n, d, ec)

    h = Wc.shape[1]
    return pl.pallas_call(
        _mlp_body,
        out_shape=jax.ShapeDtypeStruct((n, h), jnp.float32),
    )(acc, W1u, b1u, g1u, be1u, W2u, b2u, g2u, be2u,
      W1b, b1b, g1b, be1b, W2b, b2b, g2b, be2b, Wc, bc, gc, bec)


# speedup vs baseline: 12.8528x; 1.0341x over previous
import functools

import jax
import jax.numpy as jnp
from jax import lax
from jax.experimental import pallas as pl
from jax.experimental.pallas import tpu as pltpu
from jax.experimental.pallas import tpu_sc as plsc

_NC = 2
_NS = 16
_C = 128
_G = 32
_PAD_ROWS = 16


def _sc_scatter(x, up, bd, n, d, ec):
    q8, r8 = divmod(ec // 8, _NS)
    cpt_big = (q8 + 1) * 8
    cpt_small = q8 * 8
    rem_c = ec % 8
    gf_big, lo_big = divmod(cpt_big, _G)
    gf_small, lo_small = divmod(cpt_small, _G)

    rows_per_tile = (n // _NS) // 16 * 16
    tail_rows = n - _NS * rows_per_tile

    mesh = plsc.VectorSubcoreMesh(core_axis_name="c", subcore_axis_name="s")

    @functools.partial(
        pl.kernel,
        mesh=mesh,
        out_type=jax.ShapeDtypeStruct((_NC, n, d), jnp.float32),
        scratch_types=[
            pltpu.VMEM_SHARED((n + _PAD_ROWS, d), jnp.float32),
            pltpu.VMEM((2, _G * _C), jnp.int32),
            pltpu.VMEM((2, _C, d), jnp.float32),
            pltpu.SemaphoreType.DMA,
            pltpu.SemaphoreType.DMA,
            pltpu.SemaphoreType.DMA,
            pltpu.SemaphoreType.DMA,
        ],
    )
    def k(x_hbm, up_hbm, bd_hbm, out_hbm, acc_sh, idx_v, rows_v,
          sem_g0, sem_g1, sem_s0, sem_s1):
        sem_g = [sem_g0, sem_g1]
        sem_s = [sem_s0, sem_s1]
        c = lax.axis_index("c")
        s = lax.axis_index("s")
        r0 = s * rows_per_tile
        pltpu.sync_copy(x_hbm.at[pl.ds(r0, rows_per_tile)],
                        acc_sh.at[pl.ds(r0, rows_per_tile)])
        if tail_rows:
            @pl.when(s == 0)
            def _():
                t0 = _NS * rows_per_tile
                pltpu.sync_copy(x_hbm.at[pl.ds(t0, tail_rows)],
                                acc_sh.at[pl.ds(t0, tail_rows)])
        plsc.subcore_barrier()

        chunk0 = pl.multiple_of(
            jnp.where(s < r8, s * cpt_big,
                      r8 * cpt_big + (s - r8) * cpt_small), 8)

        def gather(j, b):
            return pltpu.async_copy(
                x_hbm.at[idx_v.at[0, pl.ds(j * _C, _C)]], rows_v.at[b],
                sem_g[b])

        def scatter(j, b):
            return pltpu.async_copy(
                rows_v.at[b], acc_sh.at[idx_v.at[1, pl.ds(j * _C, _C)]],
                sem_s[b], add=True)

        def pipe(idx2_hbm, row0, nchunks):
            e0 = pl.multiple_of(row0, 8) * _C
            pltpu.sync_copy(idx2_hbm.at[pl.ds(0, 2), pl.ds(e0, nchunks * _C)],
                            idx_v.at[pl.ds(0, 2), pl.ds(0, nchunks * _C)])
            desc_g = [None, None]
            desc_s = [None, None]
            for j in range(nchunks):
                b = j % 2
                if j >= 2:
                    desc_s[b].wait()
                desc_g[b] = gather(j, b)
                if j >= 1:
                    desc_g[1 - b].wait()
                    desc_s[1 - b] = scatter(j - 1, 1 - b)
            last = (nchunks - 1) % 2
            desc_g[last].wait()
            desc_s[last] = scatter(nchunks - 1, last)
            if nchunks >= 2:
                desc_s[1 - last].wait()
            desc_s[last].wait()

        gf_dyn = jnp.where(s < r8, gf_big, gf_small)

        def do_pass(idx2_hbm):
            def group(g, carry):
                pipe(idx2_hbm, chunk0 + g * _G, _G)
                return carry
            lax.fori_loop(0, gf_dyn, group, 0)
            if lo_big:
                @pl.when(s < r8)
                def _():
                    pipe(idx2_hbm, chunk0 + gf_big * _G, lo_big)
            if lo_small:
                @pl.when(s >= r8)
                def _():
                    pipe(idx2_hbm, chunk0 + gf_small * _G, lo_small)
            if rem_c:
                @pl.when(s == _NS - 1)
                def _():
                    pipe(idx2_hbm, ec - rem_c, rem_c)

        @pl.when(c == 0)
        def _():
            do_pass(up_hbm)

        @pl.when(c == 1)
        def _():
            do_pass(bd_hbm)

        plsc.subcore_barrier()
        pltpu.sync_copy(acc_sh.at[pl.ds(r0, rows_per_tile)],
                        out_hbm.at[c, pl.ds(r0, rows_per_tile)])
        if tail_rows:
            @pl.when(s == 0)
            def _():
                t0 = _NS * rows_per_tile
                pltpu.sync_copy(acc_sh.at[pl.ds(t0, tail_rows)],
                                out_hbm.at[c, pl.ds(t0, tail_rows)])

    return k(x, up, bd)


def _bn(h, g, b):
    m = jnp.mean(h, axis=0)
    v = jnp.mean(h * h, axis=0) - m * m
    return (h - m) * (lax.rsqrt(v + 1e-5) * g) + b


def _dot(a, w):
    return jnp.dot(a, w, preferred_element_type=jnp.float32)


def _mlp_body(acc_ref, W1u, b1u, g1u, be1u, W2u, b2u, g2u, be2u,
              W1b, b1b, g1b, be1b, W2b, b2b, g2b, be2b,
              Wc, bc, gc, bec, out_ref):
    h = acc_ref[0]
    hu = jax.nn.relu(_bn(_dot(h, W1u[...]) + b1u[...], g1u[...], be1u[...]))
    hu = jax.nn.relu(_bn(_dot(hu, W2u[...]) + b2u[...], g2u[...], be2u[...]))
    h = acc_ref[1]
    hb = jax.nn.relu(_bn(_dot(h, W1b[...]) + b1b[...], g1b[...], be1b[...]))
    hb = jax.nn.relu(_bn(_dot(hb, W2b[...]) + b2b[...], g2b[...], be2b[...]))
    hh = Wc.shape[0] // 2
    hc = _dot(hu, Wc[0:hh, :]) + _dot(hb, Wc[hh:, :])
    out_ref[...] = jax.nn.relu(_bn(hc + bc[...], gc[...], bec[...]))


def kernel(x, up_index, boundary_index, W1u, b1u, g1u, be1u, W2u, b2u, g2u, be2u,
           W1b, b1b, g1b, be1b, W2b, b2b, g2b, be2b, Wc, bc, gc, bec):
    n, d = x.shape
    e = up_index.shape[1]
    i32 = jnp.int32
    up = up_index.astype(i32)
    bd = boundary_index.astype(i32)
    if e % _C:
        pad = _C - e % _C
        psrc = jnp.zeros((1, pad), i32)
        pdst = jnp.full((1, pad), n, i32)
        up = jnp.concatenate([up, jnp.concatenate([psrc, pdst])], axis=1)
        bd = jnp.concatenate([bd, jnp.concatenate([psrc, pdst])], axis=1)
        e += pad
    ec = e // _C

    acc = _sc_scatter(x, up, bd, n, d, ec)

    h = Wc.shape[1]
    return pl.pallas_call(
        _mlp_body,
        out_shape=jax.ShapeDtypeStruct((n, h), jnp.float32),
    )(acc, W1u, b1u, g1u, be1u, W2u, b2u, g2u, be2u,
      W1b, b1b, g1b, be1b, W2b, b2b, g2b, be2b, Wc, bc, gc, bec)
